# Initial kernel scaffold; baseline (speedup 1.0000x reference)
import jax
import jax.numpy as jnp
from jax.experimental import pallas as pl


def kernel(x1, x2, proposal_boxes, Wc1, bc1, Wc2, bc2, Wd1, bd1, Wd2, bd2):
    raise NotImplementedError("write your pallas kernel here")


# speedup vs baseline: 737.7712x; 737.7712x over previous
import jax
import jax.numpy as jnp
from jax.experimental import pallas as pl

_N = 1200
_D = 512
_H = 42
_C = 21


def _gcn_body(x1_ref, br_ref, bc_ref, wc1_ref, bc1_ref, wc2_ref, bc2_ref,
              wd1_ref, bd1_ref, wd2_ref, bd2_ref, cls_ref, det_ref):
    x1 = x1_ref[:]

    rx1 = br_ref[:, 0:1]; ry1 = br_ref[:, 1:2]
    rx2 = br_ref[:, 2:3]; ry2 = br_ref[:, 3:4]
    cx1 = bc_ref[0:1, :]; cy1 = bc_ref[1:2, :]
    cx2 = bc_ref[2:3, :]; cy2 = bc_ref[3:4, :]
    area_r = (rx2 - rx1) * (ry2 - ry1)
    area_c = (cx2 - cx1) * (cy2 - cy1)
    iw = jnp.maximum(jnp.minimum(rx2, cx2) - jnp.maximum(rx1, cx1), 0.0)
    ih = jnp.maximum(jnp.minimum(ry2, cy2) - jnp.maximum(ry1, cy1), 0.0)
    inter = iw * ih
    union = area_r + area_c - inter
    iou = inter / jnp.maximum(union, 1e-12)
    ma = (iou >= 0.5).astype(jnp.float32)

    nrm = jnp.sqrt(jnp.sum(x1 * x1, axis=1, keepdims=True))
    xh = x1 / jnp.maximum(nrm, 1e-12)
    sim = jax.lax.dot_general(xh, xh, (((1,), (1,)), ((), ())),
                              preferred_element_type=jnp.float32)
    ms = (sim >= 0.5).astype(jnp.float32)

    rowsum = jnp.sum(x1, axis=1, keepdims=True)
    rinv = jnp.where(jnp.abs(rowsum) > 1e-12, 1.0 / rowsum, 0.0)
    x = x1 * rinv

    def conv(m, v, w, b):
        h = jnp.dot(v, w, preferred_element_type=jnp.float32)
        return jnp.dot(m, h, preferred_element_type=jnp.float32) + b

    z = jax.nn.relu(conv(ma, x, wc1_ref[:], bc1_ref[:]))
    cls = conv(ma, z, wc2_ref[:], bc2_ref[:])
    z = jax.nn.relu(conv(ms, x, wd1_ref[:], bd1_ref[:]))
    det = conv(ms, z, wd2_ref[:], bd2_ref[:])

    cls = cls - jnp.max(cls, axis=1, keepdims=True)
    ec = jnp.exp(cls)
    cls_ref[:] = ec / jnp.sum(ec, axis=1, keepdims=True)

    det = det - jnp.max(det, axis=0, keepdims=True)
    ed = jnp.exp(det)
    det_ref[:] = ed / jnp.sum(ed, axis=0, keepdims=True)


@jax.jit
def kernel(x1, x2, proposal_boxes, Wc1, bc1, Wc2, bc2, Wd1, bd1, Wd2, bd2):
    del x2
    return pl.pallas_call(
        _gcn_body,
        out_shape=(jax.ShapeDtypeStruct((_N, _C), jnp.float32),
                   jax.ShapeDtypeStruct((_N, _C), jnp.float32)),
    )(x1, proposal_boxes, proposal_boxes.T,
      Wc1, bc1.reshape(1, _H), Wc2, bc2.reshape(1, _C),
      Wd1, bd1.reshape(1, _H), Wd2, bd2.reshape(1, _C))
